# baseline (speedup 1.0000x reference)
import functools

import jax
import jax.numpy as jnp
from jax import lax
from jax.experimental import pallas as pl
from jax.experimental.pallas import tpu as pltpu
from jax.experimental.pallas import tpu_sc as plsc

F = 26
V = 100000
D = 32
B = 16384

NC = 2
NS = 16
NW = NC * NS

BPW = B // NW
NBUF = 3

VB = 12800


def _tc_relayout(tabt_ref, out_ref):
    out_ref[...] = jnp.transpose(tabt_ref[...], (0, 2, 1))


def _sc_body(idxt_hbm, tab_hbm, out_hbm, idx_v, bufs, gsem, wsem):
    wid = lax.axis_index("s") * NC + lax.axis_index("c")
    b0 = wid * BPW

    pltpu.sync_copy(idxt_hbm.at[:, pl.ds(b0, BPW)], idx_v)

    gather_handles = {}
    write_handles = {}

    def fire_gather(f):
        gather_handles[f] = pltpu.async_copy(
            tab_hbm.at[f].at[idx_v.at[f]],
            bufs[f % NBUF],
            gsem.at[f % NBUF],
        )

    for f in range(NBUF - 1):
        fire_gather(f)

    for f in range(F):
        gather_handles[f].wait()
        write_handles[f] = pltpu.async_copy(
            bufs[f % NBUF], out_hbm.at[pl.ds(b0, BPW), f], wsem.at[f % NBUF]
        )
        nxt = f + NBUF - 1
        if nxt < F:
            if f >= 1:
                write_handles[f - 1].wait()
            fire_gather(nxt)

    for f in range(F - (NBUF - 1), F):
        write_handles[f].wait()


@jax.jit
def kernel(indices, tables):
    idxt = indices.astype(jnp.int32).T

    tabt = tables.transpose(0, 2, 1)
    tab_row = pl.pallas_call(
        _tc_relayout,
        grid=(F, pl.cdiv(V, VB)),
        in_specs=[pl.BlockSpec((1, D, VB), lambda f, v: (f, 0, v))],
        out_specs=pl.BlockSpec((1, VB, D), lambda f, v: (f, v, 0)),
        out_shape=jax.ShapeDtypeStruct((F, V, D), jnp.float32),
    )(tabt)

    mesh = plsc.VectorSubcoreMesh(
        core_axis_name="c", subcore_axis_name="s",
        num_cores=NC, num_subcores=NS,
    )
    run = functools.partial(
        pl.kernel,
        mesh=mesh,
        out_type=jax.ShapeDtypeStruct((B, F, D), jnp.float32),
        scratch_types=[
            pltpu.VMEM((F, BPW), jnp.int32),
            [pltpu.VMEM((BPW, D), jnp.float32) for _ in range(NBUF)],
            pltpu.SemaphoreType.DMA((NBUF,)),
            pltpu.SemaphoreType.DMA((NBUF,)),
        ],
        compiler_params=pltpu.CompilerParams(use_tc_tiling_on_sc=False),
    )(_sc_body)
    return run(idxt, tab_row)
